# baseline (speedup 1.0000x reference)
import functools

import jax
import jax.numpy as jnp
from jax import lax
from jax.experimental import pallas as pl
from jax.experimental.pallas import tpu as pltpu
from jax.experimental.pallas import tpu_sc as plsc

NUM_BUCKETS = 32
HEADS = 16
SEQ = 2048
SCALE = 0.125
WLEN = 2 * SEQ
LANES = 16

THRESH = (16, 19, 21, 24, 27, 31, 35, 40, 46, 52, 59, 67, 77, 87, 99, 113)

R = 8
NBUF = 4
ROWS_PER_WORKER = SEQ // 2
NCHUNK = ROWS_PER_WORKER // R


def _sc_body(qk_hbm, tab_hbm, out_hbm,
             tab_v, w_v, b0, b1, b2, b3,
             si0, si1, si2, si3, so0, so1, so2, so3):
    core = lax.axis_index("c")
    head = lax.axis_index("s")

    pltpu.sync_copy(tab_hbm.at[head], tab_v)
    t_lo = tab_v[pl.ds(0, LANES)]
    t_hi = tab_v[pl.ds(LANES, LANES)]
    tvals = [t_lo[b] for b in range(LANES)] + [t_hi[b] for b in range(LANES)]

    lanes = lax.iota(jnp.int32, LANES)
    one = jnp.ones((LANES,), jnp.int32)
    zero = jnp.zeros((LANES,), jnp.int32)

    def wbody(mb, carry):
        m = mb * LANES + lanes
        n = jnp.maximum(SEQ - 1 - m, 0)
        big = jnp.zeros((LANES,), jnp.int32) + 15
        for t in THRESH:
            big = big + jnp.where(n >= t, one, zero)
        bucket = jnp.where(n < 16, n, big)
        val = jnp.zeros((LANES,), jnp.float32) + tvals[0]
        for b in range(1, NUM_BUCKETS):
            val = jnp.where(bucket == b, tvals[b], val)
        w_v[pl.ds(mb * LANES, LANES)] = val * SCALE
        return carry

    lax.fori_loop(0, WLEN // LANES, wbody, 0)

    row_base = core * ROWS_PER_WORKER
    bufs = (b0, b1, b2, b3)
    sis = (si0, si1, si2, si3)
    sos = (so0, so1, so2, so3)

    def in_slice(g):
        return qk_hbm.at[0, head, pl.ds(row_base + g * R, R), :]

    def out_slice(g):
        return out_hbm.at[0, head, pl.ds(row_base + g * R, R), :]

    pltpu.async_copy(in_slice(0), bufs[0], sis[0])

    def chunk_body(t, carry):
        for b in range(NBUF):
            g = t * NBUF + b
            bn = (b + 1) % NBUF
            ib, si, so = bufs[b], sis[b], sos[b]

            @pl.when(g + 1 < NCHUNK)
            def _prefetch():
                @pl.when(g + 1 >= NBUF)
                def _wait_prev_out():
                    pltpu.make_async_copy(
                        bufs[bn], out_slice(g + 1), sos[bn]).wait()
                pltpu.async_copy(in_slice(g + 1), bufs[bn], sis[bn])

            pltpu.make_async_copy(in_slice(g), ib, si).wait()

            off0 = SEQ - 1 - (row_base + g * R)

            @plsc.parallel_loop(0, SEQ // LANES, unroll=8)
            def cbody(ci):
                base = ci * LANES
                for r in range(R):
                    wv = w_v[pl.ds(off0 - r + base, LANES)]
                    plsc.addupdate(ib.at[r, pl.ds(base, LANES)], wv)

            pltpu.async_copy(ib, out_slice(g), so)

        return carry

    lax.fori_loop(0, NCHUNK // NBUF, chunk_body, 0)

    for b in range(NBUF):
        pltpu.make_async_copy(bufs[b], out_slice(NCHUNK - NBUF + b), sos[b]).wait()


def kernel(qk_dots, rel_bias_table):
    mesh = plsc.VectorSubcoreMesh(core_axis_name="c", subcore_axis_name="s")
    f = pl.kernel(
        _sc_body,
        out_type=jax.ShapeDtypeStruct(qk_dots.shape, qk_dots.dtype),
        mesh=mesh,
        scratch_types=[
            pltpu.VMEM((NUM_BUCKETS,), jnp.float32),
            pltpu.VMEM((WLEN,), jnp.float32),
            pltpu.VMEM((R, SEQ), jnp.float32),
            pltpu.VMEM((R, SEQ), jnp.float32),
            pltpu.VMEM((R, SEQ), jnp.float32),
            pltpu.VMEM((R, SEQ), jnp.float32),
            pltpu.SemaphoreType.DMA,
            pltpu.SemaphoreType.DMA,
            pltpu.SemaphoreType.DMA,
            pltpu.SemaphoreType.DMA,
            pltpu.SemaphoreType.DMA,
            pltpu.SemaphoreType.DMA,
            pltpu.SemaphoreType.DMA,
            pltpu.SemaphoreType.DMA,
        ],
    )
    return f(qk_dots, rel_bias_table.T)


# speedup vs baseline: 1.0145x; 1.0145x over previous
import functools

import jax
import jax.numpy as jnp
from jax import lax
from jax.experimental import pallas as pl
from jax.experimental.pallas import tpu as pltpu
from jax.experimental.pallas import tpu_sc as plsc

NUM_BUCKETS = 32
HEADS = 16
SEQ = 2048
SCALE = 0.125
WLEN = 2 * SEQ
LANES = 16

THRESH = (16, 19, 21, 24, 27, 31, 35, 40, 46, 52, 59, 67, 77, 87, 99, 113)

R = 8
NBUF = 4
ROWS_PER_WORKER = SEQ // 2
NCHUNK = ROWS_PER_WORKER // R


def _sc_body(qk_hbm, tab_hbm, out_hbm,
             tab_v, w_v, b0, b1, b2, b3,
             si0, si1, si2, si3, so0, so1, so2, so3):
    core = lax.axis_index("c")
    head = lax.axis_index("s")

    pltpu.sync_copy(tab_hbm.at[head], tab_v)
    t_lo = tab_v[pl.ds(0, LANES)]
    t_hi = tab_v[pl.ds(LANES, LANES)]
    tvals = [t_lo[b] for b in range(LANES)] + [t_hi[b] for b in range(LANES)]

    lanes = lax.iota(jnp.int32, LANES)
    one = jnp.ones((LANES,), jnp.int32)
    zero = jnp.zeros((LANES,), jnp.int32)

    row_base = core * ROWS_PER_WORKER
    bufs = (b0, b1, b2, b3)
    sis = (si0, si1, si2, si3)
    sos = (so0, so1, so2, so3)

    def in_slice(g):
        return qk_hbm.at[0, head, pl.ds(row_base + g * R, R), :]

    def out_slice(g):
        return out_hbm.at[0, head, pl.ds(row_base + g * R, R), :]

    pltpu.async_copy(in_slice(0), bufs[0], sis[0])

    @plsc.parallel_loop(0, WLEN // LANES, unroll=2)
    def wbody(mb):
        m = mb * LANES + lanes
        n = jnp.maximum(SEQ - 1 - m, 0)
        big = jnp.zeros((LANES,), jnp.int32) + 15
        for t in THRESH:
            big = big + jnp.where(n >= t, one, zero)
        bucket = jnp.where(n < 16, n, big)
        val = jnp.zeros((LANES,), jnp.float32) + tvals[0]
        for b in range(1, NUM_BUCKETS):
            val = jnp.where(bucket == b, tvals[b], val)
        w_v[pl.ds(mb * LANES, LANES)] = val * SCALE

    def chunk_body(t, carry):
        for b in range(NBUF):
            g = t * NBUF + b
            bn = (b + 1) % NBUF
            ib, si, so = bufs[b], sis[b], sos[b]

            @pl.when(g + 1 < NCHUNK)
            def _prefetch():
                @pl.when(g + 1 >= NBUF)
                def _wait_prev_out():
                    pltpu.make_async_copy(
                        bufs[bn], out_slice(g + 1), sos[bn]).wait()
                pltpu.async_copy(in_slice(g + 1), bufs[bn], sis[bn])

            pltpu.make_async_copy(in_slice(g), ib, si).wait()

            off0 = SEQ - 1 - (row_base + g * R)

            @plsc.parallel_loop(0, SEQ // LANES, unroll=4)
            def cbody(ci):
                base = ci * LANES
                for r in range(R):
                    wv = w_v[pl.ds(off0 - r + base, LANES)]
                    plsc.addupdate(ib.at[r, pl.ds(base, LANES)], wv)

            pltpu.async_copy(ib, out_slice(g), so)

        return carry

    lax.fori_loop(0, NCHUNK // NBUF, chunk_body, 0)

    for b in range(NBUF):
        pltpu.make_async_copy(bufs[b], out_slice(NCHUNK - NBUF + b), sos[b]).wait()


def kernel(qk_dots, rel_bias_table):
    mesh = plsc.VectorSubcoreMesh(core_axis_name="c", subcore_axis_name="s")
    f = pl.kernel(
        _sc_body,
        out_type=jax.ShapeDtypeStruct(qk_dots.shape, qk_dots.dtype),
        mesh=mesh,
        scratch_types=[
            pltpu.VMEM((NUM_BUCKETS,), jnp.float32),
            pltpu.VMEM((WLEN,), jnp.float32),
            pltpu.VMEM((R, SEQ), jnp.float32),
            pltpu.VMEM((R, SEQ), jnp.float32),
            pltpu.VMEM((R, SEQ), jnp.float32),
            pltpu.VMEM((R, SEQ), jnp.float32),
            pltpu.SemaphoreType.DMA,
            pltpu.SemaphoreType.DMA,
            pltpu.SemaphoreType.DMA,
            pltpu.SemaphoreType.DMA,
            pltpu.SemaphoreType.DMA,
            pltpu.SemaphoreType.DMA,
            pltpu.SemaphoreType.DMA,
            pltpu.SemaphoreType.DMA,
        ],
    )
    return f(qk_dots, rel_bias_table.T)


# speedup vs baseline: 1.0806x; 1.0652x over previous
import functools

import jax
import jax.numpy as jnp
from jax import lax
from jax.experimental import pallas as pl
from jax.experimental.pallas import tpu as pltpu
from jax.experimental.pallas import tpu_sc as plsc

NUM_BUCKETS = 32
HEADS = 16
SEQ = 2048
SCALE = 0.125
WLEN = 2 * SEQ
LANES = 16

THRESH = (16, 19, 21, 24, 27, 31, 35, 40, 46, 52, 59, 67, 77, 87, 99, 113)

R = 8
NBUF = 4
ROWS_PER_WORKER = SEQ // 2
NCHUNK = ROWS_PER_WORKER // R


def _sc_body(qk_hbm, tab_hbm, out_hbm,
             tab_v, w_v, b0, b1, b2, b3,
             si0, si1, si2, si3, so0, so1, so2, so3):
    core = lax.axis_index("c")
    head = lax.axis_index("s")

    pltpu.sync_copy(tab_hbm.at[head], tab_v)
    t_lo = tab_v[pl.ds(0, LANES)]
    t_hi = tab_v[pl.ds(LANES, LANES)]
    tvals = [t_lo[b] for b in range(LANES)] + [t_hi[b] for b in range(LANES)]

    lanes = lax.iota(jnp.int32, LANES)
    one = jnp.ones((LANES,), jnp.int32)
    zero = jnp.zeros((LANES,), jnp.int32)

    row_base = core * ROWS_PER_WORKER
    bufs = (b0, b1, b2, b3)
    sis = (si0, si1, si2, si3)
    sos = (so0, so1, so2, so3)

    def in_slice(g):
        return qk_hbm.at[0, head, pl.ds(row_base + g * R, R), :]

    def out_slice(g):
        return out_hbm.at[0, head, pl.ds(row_base + g * R, R), :]

    pltpu.async_copy(in_slice(0), bufs[0], sis[0])
    pltpu.async_copy(in_slice(1), bufs[1], sis[1])

    @plsc.parallel_loop(0, WLEN // LANES, unroll=2)
    def wbody(mb):
        m = mb * LANES + lanes
        n = jnp.maximum(SEQ - 1 - m, 0)
        big = jnp.zeros((LANES,), jnp.int32) + 15
        for t in THRESH:
            big = big + jnp.where(n >= t, one, zero)
        bucket = jnp.where(n < 16, n, big)
        val = jnp.zeros((LANES,), jnp.float32) + tvals[0]
        for b in range(1, NUM_BUCKETS):
            val = jnp.where(bucket == b, tvals[b], val)
        w_v[pl.ds(mb * LANES, LANES)] = val * SCALE

    def chunk_body(t, carry):
        for b in range(NBUF):
            g = t * NBUF + b
            bn = (b + 2) % NBUF
            ib, si, so = bufs[b], sis[b], sos[b]

            @pl.when(g + 2 < NCHUNK)
            def _prefetch():
                @pl.when(g + 2 >= NBUF)
                def _wait_prev_out():
                    pltpu.make_async_copy(
                        bufs[bn], out_slice(g + 2), sos[bn]).wait()
                pltpu.async_copy(in_slice(g + 2), bufs[bn], sis[bn])

            pltpu.make_async_copy(in_slice(g), ib, si).wait()

            off0 = SEQ - 1 - (row_base + g * R)

            @plsc.parallel_loop(0, SEQ // LANES, unroll=4)
            def cbody(ci):
                base = ci * LANES
                for r in range(R):
                    wv = w_v[pl.ds(off0 - r + base, LANES)]
                    plsc.addupdate(ib.at[r, pl.ds(base, LANES)], wv)

            pltpu.async_copy(ib, out_slice(g), so)

        return carry

    lax.fori_loop(0, NCHUNK // NBUF, chunk_body, 0)

    for b in range(NBUF):
        pltpu.make_async_copy(bufs[b], out_slice(NCHUNK - NBUF + b), sos[b]).wait()


def kernel(qk_dots, rel_bias_table):
    mesh = plsc.VectorSubcoreMesh(core_axis_name="c", subcore_axis_name="s")
    f = pl.kernel(
        _sc_body,
        out_type=jax.ShapeDtypeStruct(qk_dots.shape, qk_dots.dtype),
        mesh=mesh,
        scratch_types=[
            pltpu.VMEM((NUM_BUCKETS,), jnp.float32),
            pltpu.VMEM((WLEN,), jnp.float32),
            pltpu.VMEM((R, SEQ), jnp.float32),
            pltpu.VMEM((R, SEQ), jnp.float32),
            pltpu.VMEM((R, SEQ), jnp.float32),
            pltpu.VMEM((R, SEQ), jnp.float32),
            pltpu.SemaphoreType.DMA,
            pltpu.SemaphoreType.DMA,
            pltpu.SemaphoreType.DMA,
            pltpu.SemaphoreType.DMA,
            pltpu.SemaphoreType.DMA,
            pltpu.SemaphoreType.DMA,
            pltpu.SemaphoreType.DMA,
            pltpu.SemaphoreType.DMA,
        ],
    )
    return f(qk_dots, rel_bias_table.T)
